# baseline (speedup 1.0000x reference)
import functools

import jax
import jax.numpy as jnp
from jax import lax
from jax.experimental import pallas as pl
from jax.experimental.pallas import tpu as pltpu
from jax.experimental.pallas import tpu_sc as plsc


def _table_body(emb_ref, w_ref, b_ref, g_ref, beta_ref, out_ref):
    h = lax.dot_general(
        emb_ref[...], w_ref[...], (((1,), (1,)), ((), ())),
        preferred_element_type=jnp.float32,
    )
    h = h + b_ref[...]
    mu = jnp.mean(h, axis=-1, keepdims=True)
    d = h - mu
    var = jnp.mean(d * d, axis=-1, keepdims=True)
    out_ref[...] = (d * lax.rsqrt(var + 1e-5)) * g_ref[...] + beta_ref[...]


def _make_table(emb, W, b, gamma, beta):
    V, D = emb.shape
    return pl.pallas_call(
        _table_body,
        out_shape=jax.ShapeDtypeStruct((V, D), jnp.float32),
    )(emb, W, b.reshape(1, D), gamma.reshape(1, D), beta.reshape(1, D))


_NSLOT = 2
_CHUNK = 400


def _make_expand(B, L, V, D):
    N2 = (B * L) // 2
    pr_per_w = N2 // 32
    chunk_pr = _CHUNK
    n_outer = pr_per_w // (chunk_pr * _NSLOT)
    pr_per_sc = N2 // 2
    mesh = plsc.VectorSubcoreMesh(core_axis_name="c", subcore_axis_name="s")

    scratch = (
        [pltpu.VMEM((V * D,), jnp.float32),
         pltpu.VMEM_SHARED((pr_per_sc,), jnp.int32),
         pltpu.SMEM((_NSLOT * chunk_pr,), jnp.int32)]
        + [pltpu.VMEM((chunk_pr * 2 * D,), jnp.float32) for _ in range(_NSLOT)]
        + [pltpu.SemaphoreType.DMA for _ in range(_NSLOT + 1)]
    )

    @functools.partial(
        pl.kernel,
        out_type=jax.ShapeDtypeStruct((B * L * D,), jnp.float32),
        mesh=mesh,
        scratch_types=scratch,
        compiler_params=pltpu.CompilerParams(needs_layout_passes=False),
    )
    def expand_k(tab_hbm, pidx_hbm, out_hbm, *refs):
        tab_v = refs[0]
        p_sh = refs[1]
        p_sm = refs[2]
        bufs = refs[3:3 + _NSLOT]
        sem_s = refs[3 + _NSLOT:3 + 2 * _NSLOT]
        sem_ld = refs[3 + 2 * _NSLOT]

        cid = lax.axis_index("c")
        sid = lax.axis_index("s")
        wid = cid * 16 + sid
        pr0 = pl.multiple_of(wid * pr_per_w, pr_per_w)

        pltpu.async_copy(tab_hbm, tab_v, sem_ld)

        @pl.when(sid == 0)
        def _():
            r0 = pl.multiple_of(cid * pr_per_sc, pr_per_sc)
            pltpu.sync_copy(pidx_hbm.at[pl.ds(r0, pr_per_sc)], p_sh)

        pltpu.make_async_copy(tab_hbm, tab_v, sem_ld).wait()
        plsc.subcore_barrier()

        def expand_chunk(sm_p0, buf):
            @plsc.parallel_loop(0, chunk_pr, 1, unroll=2)
            def pair(p):
                pidx = p_sm[sm_p0 + p]
                tb0 = (pidx >> 4) * D
                tb1 = (pidx & 15) * D
                ob = p * 2 * D
                for k in range(D // 16):
                    buf[pl.ds(ob + k * 16, 16)] = tab_v[pl.ds(tb0 + k * 16, 16)]
                for k in range(D // 16):
                    buf[pl.ds(ob + D + k * 16, 16)] = (
                        tab_v[pl.ds(tb1 + k * 16, 16)]
                    )

        def fire_store(g, b):
            off = (pr0 + g * chunk_pr) * 2 * D
            pltpu.async_copy(
                bufs[b], out_hbm.at[pl.ds(off, chunk_pr * 2 * D)], sem_s[b]
            )

        def wait_store(b):
            pltpu.make_async_copy(
                bufs[b], out_hbm.at[pl.ds(pr0 * 2 * D, chunk_pr * 2 * D)],
                sem_s[b],
            ).wait()

        def outer(i, carry):
            sp0 = sid * pr_per_w + i * _NSLOT * chunk_pr
            pltpu.sync_copy(
                p_sh.at[pl.ds(sp0, _NSLOT * chunk_pr)], p_sm
            )
            for b in range(_NSLOT):
                g = i * _NSLOT + b

                @pl.when(i > 0)
                def _():
                    wait_store(b)

                expand_chunk(b * chunk_pr, bufs[b])
                fire_store(g, b)
            return carry

        lax.fori_loop(0, n_outer, outer, 0)
        for b in range(_NSLOT):
            wait_store(b)

    return expand_k


def kernel(x, emb, W, b, gamma, beta):
    B, L = x.shape
    V, D = emb.shape
    xi = x.astype(jnp.int32)
    pidx = (xi[:, 0::2] * V + xi[:, 1::2]).reshape(-1)
    table = _make_table(emb, W, b, gamma, beta).reshape(-1)
    expand = _make_expand(B, L, V, D)
    out = expand(table, pidx)
    return out.reshape(B, L, D)


# speedup vs baseline: 1.3213x; 1.3213x over previous
import functools

import jax
import jax.numpy as jnp
from jax import lax
from jax.experimental import pallas as pl
from jax.experimental.pallas import tpu as pltpu
from jax.experimental.pallas import tpu_sc as plsc


def _table_body(emb_ref, w_ref, b_ref, g_ref, beta_ref, out_ref):
    h = lax.dot_general(
        emb_ref[...], w_ref[...], (((1,), (1,)), ((), ())),
        preferred_element_type=jnp.float32,
    )
    h = h + b_ref[...]
    mu = jnp.mean(h, axis=-1, keepdims=True)
    d = h - mu
    var = jnp.mean(d * d, axis=-1, keepdims=True)
    out_ref[...] = (d * lax.rsqrt(var + 1e-5)) * g_ref[...] + beta_ref[...]


def _make_table(emb, W, b, gamma, beta):
    V, D = emb.shape
    return pl.pallas_call(
        _table_body,
        out_shape=jax.ShapeDtypeStruct((V, D), jnp.float32),
    )(emb, W, b.reshape(1, D), gamma.reshape(1, D), beta.reshape(1, D))


_NSLOT = 2
_RPC = 2


def _make_expand(B, L, V, D):
    L2 = L // 2
    N2 = (B * L) // 2
    pr_per_w = N2 // 32
    rows_per_w = B // 32
    chunk_pr = _RPC * L2
    n_outer = rows_per_w // (_RPC * _NSLOT)
    pr_per_sc = N2 // 2
    mesh = plsc.VectorSubcoreMesh(core_axis_name="c", subcore_axis_name="s")

    scratch = (
        [pltpu.VMEM((V * D,), jnp.float32),
         pltpu.VMEM_SHARED((pr_per_sc,), jnp.int32),
         pltpu.SMEM((_NSLOT * chunk_pr,), jnp.int32)]
        + [pltpu.VMEM((_RPC, L, D), jnp.float32) for _ in range(_NSLOT)]
        + [pltpu.SemaphoreType.DMA for _ in range(_NSLOT + 1)]
    )

    @functools.partial(
        pl.kernel,
        out_type=jax.ShapeDtypeStruct((B, L, D), jnp.float32),
        mesh=mesh,
        scratch_types=scratch,
        compiler_params=pltpu.CompilerParams(needs_layout_passes=False),
    )
    def expand_k(tab_hbm, pidx_hbm, out_hbm, *refs):
        tab_v = refs[0]
        p_sh = refs[1]
        p_sm = refs[2]
        bufs = refs[3:3 + _NSLOT]
        sem_s = refs[3 + _NSLOT:3 + 2 * _NSLOT]
        sem_ld = refs[3 + 2 * _NSLOT]

        cid = lax.axis_index("c")
        sid = lax.axis_index("s")
        wid = cid * 16 + sid
        row0 = pl.multiple_of(wid * rows_per_w, rows_per_w)

        pltpu.async_copy(tab_hbm, tab_v, sem_ld)

        @pl.when(sid == 0)
        def _():
            r0 = pl.multiple_of(cid * pr_per_sc, pr_per_sc)
            pltpu.sync_copy(pidx_hbm.at[pl.ds(r0, pr_per_sc)], p_sh)

        pltpu.make_async_copy(tab_hbm, tab_v, sem_ld).wait()
        plsc.subcore_barrier()

        def expand_chunk(sm_p0, buf):
            @plsc.parallel_loop(0, L2, 1, unroll=2)
            def pair(j):
                for r in range(_RPC):
                    pidx = p_sm[sm_p0 + r * L2 + j]
                    tb0 = (pidx >> 4) * D
                    tb1 = (pidx & 15) * D
                    for k in range(D // 16):
                        buf[r, 2 * j, pl.ds(k * 16, 16)] = (
                            tab_v[pl.ds(tb0 + k * 16, 16)]
                        )
                    for k in range(D // 16):
                        buf[r, 2 * j + 1, pl.ds(k * 16, 16)] = (
                            tab_v[pl.ds(tb1 + k * 16, 16)]
                        )

        def fire_store(g, b):
            r = row0 + g * _RPC
            pltpu.async_copy(
                bufs[b], out_hbm.at[pl.ds(r, _RPC), :, :], sem_s[b]
            )

        def wait_store(b):
            pltpu.make_async_copy(
                bufs[b], out_hbm.at[pl.ds(row0, _RPC), :, :], sem_s[b]
            ).wait()

        def outer(i, carry):
            sp0 = sid * pr_per_w + i * _NSLOT * chunk_pr
            pltpu.sync_copy(
                p_sh.at[pl.ds(sp0, _NSLOT * chunk_pr)], p_sm
            )
            for b in range(_NSLOT):
                g = i * _NSLOT + b

                @pl.when(i > 0)
                def _():
                    wait_store(b)

                expand_chunk(b * chunk_pr, bufs[b])
                fire_store(g, b)
            return carry

        lax.fori_loop(0, n_outer, outer, 0)
        for b in range(_NSLOT):
            wait_store(b)

    return expand_k


def kernel(x, emb, W, b, gamma, beta):
    B, L = x.shape
    V, D = emb.shape
    xi = x.astype(jnp.int32)
    pidx = (xi[:, 0::2] * V + xi[:, 1::2]).reshape(-1)
    table = _make_table(emb, W, b, gamma, beta).reshape(-1)
    expand = _make_expand(B, L, V, D)
    return expand(table, pidx)


# speedup vs baseline: 1.7102x; 1.2944x over previous
import functools

import jax
import jax.numpy as jnp
from jax import lax
from jax.experimental import pallas as pl
from jax.experimental.pallas import tpu as pltpu
from jax.experimental.pallas import tpu_sc as plsc


def _table_body(emb_ref, w_ref, b_ref, g_ref, beta_ref, out_ref):
    h = lax.dot_general(
        emb_ref[...], w_ref[...], (((1,), (1,)), ((), ())),
        preferred_element_type=jnp.float32,
    )
    h = h + b_ref[...]
    mu = jnp.mean(h, axis=-1, keepdims=True)
    d = h - mu
    var = jnp.mean(d * d, axis=-1, keepdims=True)
    out_ref[...] = (d * lax.rsqrt(var + 1e-5)) * g_ref[...] + beta_ref[...]


def _make_table(emb, W, b, gamma, beta):
    V, D = emb.shape
    return pl.pallas_call(
        _table_body,
        out_shape=jax.ShapeDtypeStruct((V, D), jnp.float32),
    )(emb, W, b.reshape(1, D), gamma.reshape(1, D), beta.reshape(1, D))


_NSLOT = 2
_RPC = 2


def _make_expand(B, L, V, D):
    L2 = L // 2
    N2 = (B * L) // 2
    pr_per_w = N2 // 32
    rows_per_w = B // 32
    chunk_pr = _RPC * L2
    n_outer = rows_per_w // (_RPC * _NSLOT)
    pr_per_sc = N2 // 2
    mesh = plsc.VectorSubcoreMesh(core_axis_name="c", subcore_axis_name="s")

    scratch = (
        [pltpu.VMEM((V * D,), jnp.float32),
         pltpu.VMEM_SHARED((pr_per_sc,), jnp.int32),
         pltpu.SMEM((_NSLOT * chunk_pr,), jnp.int32)]
        + [pltpu.VMEM((_RPC, L, D), jnp.float32) for _ in range(_NSLOT)]
        + [pltpu.SemaphoreType.DMA for _ in range(_NSLOT + 1)]
    )

    @functools.partial(
        pl.kernel,
        out_type=jax.ShapeDtypeStruct((B, L, D), jnp.float32),
        mesh=mesh,
        scratch_types=scratch,
        compiler_params=pltpu.CompilerParams(needs_layout_passes=False),
    )
    def expand_k(tab_hbm, pidx_hbm, out_hbm, *refs):
        tab_v = refs[0]
        p_sh = refs[1]
        p_sm = refs[2]
        bufs = refs[3:3 + _NSLOT]
        sem_s = refs[3 + _NSLOT:3 + 2 * _NSLOT]
        sem_ld = refs[3 + 2 * _NSLOT]

        cid = lax.axis_index("c")
        sid = lax.axis_index("s")
        wid = cid * 16 + sid
        row0 = pl.multiple_of(wid * rows_per_w, rows_per_w)

        pltpu.async_copy(tab_hbm, tab_v, sem_ld)

        @pl.when(sid == 0)
        def _():
            r0 = pl.multiple_of(cid * pr_per_sc, pr_per_sc)
            pltpu.sync_copy(pidx_hbm.at[pl.ds(r0, pr_per_sc)], p_sh)

        pltpu.make_async_copy(tab_hbm, tab_v, sem_ld).wait()
        plsc.subcore_barrier()

        def expand_chunk(sm_p0, buf):
            @plsc.parallel_loop(0, L2, 1, unroll=2)
            def pair(j):
                for r in range(_RPC):
                    pidx = p_sm[sm_p0 + r * L2 + j]
                    tb0 = (pidx >> 4) * D
                    tb1 = (pidx & 15) * D
                    for k in range(D // 16):
                        buf[r, 2 * j, pl.ds(k * 16, 16)] = (
                            tab_v[pl.ds(tb0 + k * 16, 16)]
                        )
                    for k in range(D // 16):
                        buf[r, 2 * j + 1, pl.ds(k * 16, 16)] = (
                            tab_v[pl.ds(tb1 + k * 16, 16)]
                        )

        def fire_store(g, b):
            r = row0 + g * _RPC
            pltpu.async_copy(
                bufs[b], out_hbm.at[pl.ds(r, _RPC), :, :], sem_s[b]
            )

        def wait_store(b):
            pltpu.make_async_copy(
                bufs[b], out_hbm.at[pl.ds(row0, _RPC), :, :], sem_s[b]
            ).wait()

        def outer(i, carry):
            sp0 = sid * pr_per_w + i * _NSLOT * chunk_pr
            pltpu.sync_copy(
                p_sh.at[pl.ds(sp0, _NSLOT * chunk_pr)], p_sm
            )
            for b in range(_NSLOT):
                g = i * _NSLOT + b

                @pl.when(i > 0)
                def _():
                    wait_store(b)

                expand_chunk(b * chunk_pr, bufs[b])
                fire_store(g, b)
            return carry

        lax.fori_loop(0, n_outer, outer, 0)
        for b in range(_NSLOT):
            wait_store(b)

    return expand_k


def kernel(x, emb, W, b, gamma, beta):
    B, L = x.shape
    V, D = emb.shape
    xi = x.astype(jnp.int32)
    pidx = (xi[:, 0::2] * V + xi[:, 1::2]).reshape(-1)
    table = _make_table(emb, W, b, gamma, beta).reshape(-1)
    expand = _make_expand(B, L, V, D)
    return lax.optimization_barrier(expand(table, pidx))
